# Initial kernel scaffold; baseline (speedup 1.0000x reference)
import jax
import jax.numpy as jnp
from jax.experimental import pallas as pl


def kernel(x, edge_index, W1, b1, W2, b2, Wc, bc):
    raise NotImplementedError("write your pallas kernel here")


# speedup vs baseline: 30.0802x; 30.0802x over previous
import functools

import jax
import jax.numpy as jnp
from jax import lax
from jax.experimental import pallas as pl
from jax.experimental.pallas import tpu as pltpu
from jax.experimental.pallas import tpu_sc as plsc

NC = 2
NS = 16
CH = 128

f32 = jnp.float32
i32 = jnp.int32


def _mesh():
    return plsc.VectorSubcoreMesh(core_axis_name="c", subcore_axis_name="s",
                                  num_cores=NC, num_subcores=NS)


def _deg_body(colp, zeros, out, idx_v, ones_v, deg_sh, *, n_pad, k):
    c = lax.axis_index("c")
    s = lax.axis_index("s")
    sl = n_pad // NS
    pltpu.sync_copy(zeros.at[pl.ds(s * sl, sl)], deg_sh.at[pl.ds(s * sl, sl)])
    pltpu.sync_copy(colp.at[c].at[s], idx_v)
    for i in range(CH // 16):
        ones_v[pl.ds(i * 16, 16)] = jnp.full((16,), 1.0, f32)
    plsc.subcore_barrier()

    def body(j, _):
        pltpu.sync_copy(ones_v, deg_sh.at[idx_v.at[j]], add=True)
        return ()

    lax.fori_loop(0, k, body, ())
    plsc.subcore_barrier()
    pltpu.sync_copy(deg_sh.at[pl.ds(s * sl, sl)],
                    out.at[c].at[pl.ds(s * sl, sl)])


def _seg_body(yp, gidx, sidx, zeros, out, gidx_v, sidx_v, msg_v, y_sh, s_sh,
              *, n_pad, k4):
    c = lax.axis_index("c")
    s = lax.axis_index("s")
    sl = (n_pad * 4) // NS
    pltpu.sync_copy(zeros.at[pl.ds(s * sl, sl)], s_sh.at[pl.ds(s * sl, sl)])
    pltpu.sync_copy(yp.at[pl.ds(s * sl, sl)], y_sh.at[pl.ds(s * sl, sl)])
    for f in range(4):
        pltpu.sync_copy(gidx.at[f].at[c].at[s], gidx_v.at[f])
        pltpu.sync_copy(sidx.at[f].at[c].at[s], sidx_v.at[f])
    plsc.subcore_barrier()

    for f in range(4):
        def body(j, _):
            pltpu.sync_copy(y_sh.at[gidx_v.at[f].at[j]], msg_v)
            pltpu.sync_copy(msg_v, s_sh.at[sidx_v.at[f].at[j]], add=True)
            return ()

        lax.fori_loop(0, k4 // 4, body, ())
    plsc.subcore_barrier()
    pltpu.sync_copy(s_sh.at[pl.ds(s * sl, sl)],
                    out.at[c].at[pl.ds(s * sl, sl)])


def _tc1_body(x_ref, w1_ref, degp_ref, y_ref, dinv_ref):
    deg = degp_ref[0] + degp_ref[1] + 1.0
    dv = lax.rsqrt(deg)
    xw = jnp.dot(x_ref[...], w1_ref[...], preferred_element_type=f32)
    y_ref[...] = xw * dv
    dinv_ref[...] = dv


def _tc2_body(sp_ref, y_ref, dinv_ref, b_ref, w_ref, y2_ref):
    s = sp_ref[0] + sp_ref[1]
    dv = dinv_ref[...]
    h = jnp.tanh(dv * (s + y_ref[...]) + b_ref[...])
    y2_ref[...] = jnp.dot(h, w_ref[...], preferred_element_type=f32) * dv


def _tc3_body(sp_ref, y_ref, dinv_ref, b_ref, wc_ref, bc_ref,
              out_ref, h_ref):
    s = sp_ref[0] + sp_ref[1]
    dv = dinv_ref[...]
    h = jnp.tanh(dv * (s + y_ref[...]) + b_ref[...])
    h_ref[...] = h
    out_ref[...] = jnp.dot(h, wc_ref[...], preferred_element_type=f32) + bc_ref[...]


def kernel(x, edge_index, W1, b1, W2, b2, Wc, bc):
    n, d = x.shape
    h_dim = W1.shape[1]
    c_dim = Wc.shape[1]
    e = edge_index.shape[1]

    n_pad = ((n // (NS * 128)) + 1) * (NS * 128)
    k = -(-e // (NC * NS * CH))
    e_pad = NC * NS * k * CH

    row = edge_index[0]
    col = edge_index[1]
    pad = e_pad - e
    colp = jnp.concatenate(
        [col, n + (jnp.arange(pad, dtype=i32) % (n_pad - n))])
    colp = colp.reshape(NC, NS, k, CH)

    rowp = jnp.concatenate(
        [row, (jnp.arange(pad, dtype=i32) % n)]).reshape(NC, NS, k, CH)
    four = jnp.arange(4, dtype=i32).reshape(4, 1, 1, 1, 1)
    gidx = rowp[None] * 4 + four
    sidx = colp[None] * 4 + four
    k4 = 4 * k

    zeros1 = jnp.zeros((n_pad,), f32)
    zeros4 = jnp.zeros((n_pad * 4,), f32)

    mesh = _mesh()

    deg_call = pl.kernel(
        functools.partial(_deg_body, n_pad=n_pad, k=k),
        out_type=jax.ShapeDtypeStruct((NC, n_pad), f32),
        mesh=mesh,
        scratch_types=[
            pltpu.VMEM((k, CH), i32),
            pltpu.VMEM((CH,), f32),
            pltpu.VMEM_SHARED((n_pad,), f32),
        ],
    )
    degp = deg_call(colp, zeros1)

    seg_call = pl.kernel(
        functools.partial(_seg_body, n_pad=n_pad, k4=k4),
        out_type=jax.ShapeDtypeStruct((NC, n_pad * 4), f32),
        mesh=mesh,
        scratch_types=[
            pltpu.VMEM((4, k, CH), i32),
            pltpu.VMEM((4, k, CH), i32),
            pltpu.VMEM((CH,), f32),
            pltpu.VMEM_SHARED((n_pad * 4,), f32),
            pltpu.VMEM_SHARED((n_pad * 4,), f32),
        ],
    )

    bm = 2000
    grid = n // bm
    degp3 = degp.reshape(NC, n_pad, 1)

    y1, dinv = pl.pallas_call(
        _tc1_body,
        grid=(grid,),
        in_specs=[
            pl.BlockSpec((bm, d), lambda i: (i, 0)),
            pl.BlockSpec((d, h_dim), lambda i: (0, 0)),
            pl.BlockSpec((NC, bm, 1), lambda i: (0, i, 0)),
        ],
        out_specs=[
            pl.BlockSpec((bm, h_dim), lambda i: (i, 0)),
            pl.BlockSpec((bm, 1), lambda i: (i, 0)),
        ],
        out_shape=[
            jax.ShapeDtypeStruct((n_pad, h_dim), f32),
            jax.ShapeDtypeStruct((n, 1), f32),
        ],
    )(x, W1, degp3)

    s1p = seg_call(y1.reshape(-1), gidx, sidx, zeros4).reshape(NC, n_pad, 4)

    y2 = pl.pallas_call(
        _tc2_body,
        grid=(grid,),
        in_specs=[
            pl.BlockSpec((NC, bm, 4), lambda i: (0, i, 0)),
            pl.BlockSpec((bm, h_dim), lambda i: (i, 0)),
            pl.BlockSpec((bm, 1), lambda i: (i, 0)),
            pl.BlockSpec((1, h_dim), lambda i: (0, 0)),
            pl.BlockSpec((h_dim, h_dim), lambda i: (0, 0)),
        ],
        out_specs=pl.BlockSpec((bm, h_dim), lambda i: (i, 0)),
        out_shape=jax.ShapeDtypeStruct((n_pad, h_dim), f32),
    )(s1p, y1, dinv, b1.reshape(1, h_dim), W2)

    s2p = seg_call(y2.reshape(-1), gidx, sidx, zeros4).reshape(NC, n_pad, 4)

    out, h2 = pl.pallas_call(
        _tc3_body,
        grid=(grid,),
        in_specs=[
            pl.BlockSpec((NC, bm, 4), lambda i: (0, i, 0)),
            pl.BlockSpec((bm, h_dim), lambda i: (i, 0)),
            pl.BlockSpec((bm, 1), lambda i: (i, 0)),
            pl.BlockSpec((1, h_dim), lambda i: (0, 0)),
            pl.BlockSpec((h_dim, c_dim), lambda i: (0, 0)),
            pl.BlockSpec((1, c_dim), lambda i: (0, 0)),
        ],
        out_specs=[
            pl.BlockSpec((bm, c_dim), lambda i: (i, 0)),
            pl.BlockSpec((bm, h_dim), lambda i: (i, 0)),
        ],
        out_shape=[
            jax.ShapeDtypeStruct((n, c_dim), f32),
            jax.ShapeDtypeStruct((n, h_dim), f32),
        ],
    )(s2p, y2, dinv, b2.reshape(1, h_dim), Wc, bc.reshape(1, c_dim))

    return (out, h2)


# speedup vs baseline: 38.0036x; 1.2634x over previous
import functools

import jax
import jax.numpy as jnp
from jax import lax
from jax.experimental import pallas as pl
from jax.experimental.pallas import tpu as pltpu
from jax.experimental.pallas import tpu_sc as plsc

NC = 2
NS = 16
CH = 128

f32 = jnp.float32
i32 = jnp.int32


def _mesh():
    return plsc.VectorSubcoreMesh(core_axis_name="c", subcore_axis_name="s",
                                  num_cores=NC, num_subcores=NS)


def _deg_body(colp, zeros, out, idx_v, ones_v, deg_sh, *, n_pad, k):
    c = lax.axis_index("c")
    s = lax.axis_index("s")
    sl = n_pad // NS
    pltpu.sync_copy(zeros.at[pl.ds(s * sl, sl)], deg_sh.at[pl.ds(s * sl, sl)])
    pltpu.sync_copy(colp.at[c].at[s], idx_v)
    for i in range(CH // 16):
        ones_v[pl.ds(i * 16, 16)] = jnp.full((16,), 1.0, f32)
    plsc.subcore_barrier()

    def body(j, _):
        pltpu.sync_copy(ones_v, deg_sh.at[idx_v.at[j]], add=True)
        return ()

    lax.fori_loop(0, k, body, ())
    plsc.subcore_barrier()
    pltpu.sync_copy(deg_sh.at[pl.ds(s * sl, sl)],
                    out.at[c].at[pl.ds(s * sl, sl)])


def _seg_body(yp, gidx, sidx, zeros, out, gidx_v, sidx_v, msg_v, y_sh, s_sh,
              gsem, ssem, *, n_pad, k4):
    c = lax.axis_index("c")
    s = lax.axis_index("s")
    k = k4 // 4
    sl = (n_pad * 4) // NS
    pltpu.sync_copy(zeros.at[pl.ds(s * sl, sl)], s_sh.at[pl.ds(s * sl, sl)])
    pltpu.sync_copy(yp.at[pl.ds(s * sl, sl)], y_sh.at[pl.ds(s * sl, sl)])
    for f in range(4):
        pltpu.sync_copy(gidx.at[f].at[c].at[s], gidx_v.at[f])
        pltpu.sync_copy(sidx.at[f].at[c].at[s], sidx_v.at[f])
    plsc.subcore_barrier()

    def gidx_at(j):
        return gidx_v.at[j // k].at[lax.rem(j, k)]

    def sidx_at(j):
        return sidx_v.at[j // k].at[lax.rem(j, k)]

    pltpu.async_copy(y_sh.at[gidx_at(0)], msg_v.at[0], gsem)

    def body(j, _):
        b = lax.rem(j, 2)

        @pl.when(j >= 1)
        def _():
            pltpu.make_async_copy(
                msg_v.at[1 - b], s_sh.at[sidx_at(j - 1)], ssem).wait()

        @pl.when(j + 1 < k4)
        def _():
            pltpu.async_copy(y_sh.at[gidx_at(j + 1)], msg_v.at[1 - b], gsem)

        pltpu.make_async_copy(y_sh.at[gidx_at(j)], msg_v.at[b], gsem).wait()
        pltpu.async_copy(msg_v.at[b], s_sh.at[sidx_at(j)], ssem, add=True)
        return ()

    lax.fori_loop(0, k4, body, ())
    pltpu.make_async_copy(
        msg_v.at[lax.rem(k4 - 1, 2)], s_sh.at[sidx_at(k4 - 1)], ssem).wait()
    plsc.subcore_barrier()
    pltpu.sync_copy(s_sh.at[pl.ds(s * sl, sl)],
                    out.at[c].at[pl.ds(s * sl, sl)])


def _tc1_body(x_ref, w1_ref, degp_ref, y_ref, dinv_ref):
    deg = degp_ref[0] + degp_ref[1] + 1.0
    dv = lax.rsqrt(deg)
    xw = jnp.dot(x_ref[...], w1_ref[...], preferred_element_type=f32)
    y_ref[...] = xw * dv
    dinv_ref[...] = dv


def _tc2_body(sp_ref, y_ref, dinv_ref, b_ref, w_ref, y2_ref):
    s = sp_ref[0] + sp_ref[1]
    dv = dinv_ref[...]
    h = jnp.tanh(dv * (s + y_ref[...]) + b_ref[...])
    y2_ref[...] = jnp.dot(h, w_ref[...], preferred_element_type=f32) * dv


def _tc3_body(sp_ref, y_ref, dinv_ref, b_ref, wc_ref, bc_ref,
              out_ref, h_ref):
    s = sp_ref[0] + sp_ref[1]
    dv = dinv_ref[...]
    h = jnp.tanh(dv * (s + y_ref[...]) + b_ref[...])
    h_ref[...] = h
    out_ref[...] = jnp.dot(h, wc_ref[...], preferred_element_type=f32) + bc_ref[...]


def kernel(x, edge_index, W1, b1, W2, b2, Wc, bc):
    n, d = x.shape
    h_dim = W1.shape[1]
    c_dim = Wc.shape[1]
    e = edge_index.shape[1]

    n_pad = ((n // (NS * 128)) + 1) * (NS * 128)
    k = -(-e // (NC * NS * CH))
    e_pad = NC * NS * k * CH

    row = edge_index[0]
    col = edge_index[1]
    pad = e_pad - e
    colp = jnp.concatenate(
        [col, n + (jnp.arange(pad, dtype=i32) % (n_pad - n))])
    colp = colp.reshape(NC, NS, k, CH)

    rowp = jnp.concatenate(
        [row, (jnp.arange(pad, dtype=i32) % n)]).reshape(NC, NS, k, CH)
    four = jnp.arange(4, dtype=i32).reshape(4, 1, 1, 1, 1)
    gidx = rowp[None] * 4 + four
    sidx = colp[None] * 4 + four
    k4 = 4 * k

    zeros1 = jnp.zeros((n_pad,), f32)
    zeros4 = jnp.zeros((n_pad * 4,), f32)

    mesh = _mesh()

    deg_call = pl.kernel(
        functools.partial(_deg_body, n_pad=n_pad, k=k),
        out_type=jax.ShapeDtypeStruct((NC, n_pad), f32),
        mesh=mesh,
        scratch_types=[
            pltpu.VMEM((k, CH), i32),
            pltpu.VMEM((CH,), f32),
            pltpu.VMEM_SHARED((n_pad,), f32),
        ],
    )
    degp = deg_call(colp, zeros1)

    seg_call = pl.kernel(
        functools.partial(_seg_body, n_pad=n_pad, k4=k4),
        out_type=jax.ShapeDtypeStruct((NC, n_pad * 4), f32),
        mesh=mesh,
        scratch_types=[
            pltpu.VMEM((4, k, CH), i32),
            pltpu.VMEM((4, k, CH), i32),
            pltpu.VMEM((2, CH), f32),
            pltpu.VMEM_SHARED((n_pad * 4,), f32),
            pltpu.VMEM_SHARED((n_pad * 4,), f32),
            pltpu.SemaphoreType.DMA,
            pltpu.SemaphoreType.DMA,
        ],
    )

    bm = 2000
    grid = n // bm
    degp3 = degp.reshape(NC, n_pad, 1)

    y1, dinv = pl.pallas_call(
        _tc1_body,
        grid=(grid,),
        in_specs=[
            pl.BlockSpec((bm, d), lambda i: (i, 0)),
            pl.BlockSpec((d, h_dim), lambda i: (0, 0)),
            pl.BlockSpec((NC, bm, 1), lambda i: (0, i, 0)),
        ],
        out_specs=[
            pl.BlockSpec((bm, h_dim), lambda i: (i, 0)),
            pl.BlockSpec((bm, 1), lambda i: (i, 0)),
        ],
        out_shape=[
            jax.ShapeDtypeStruct((n_pad, h_dim), f32),
            jax.ShapeDtypeStruct((n, 1), f32),
        ],
    )(x, W1, degp3)

    s1p = seg_call(y1.reshape(-1), gidx, sidx, zeros4).reshape(NC, n_pad, 4)

    y2 = pl.pallas_call(
        _tc2_body,
        grid=(grid,),
        in_specs=[
            pl.BlockSpec((NC, bm, 4), lambda i: (0, i, 0)),
            pl.BlockSpec((bm, h_dim), lambda i: (i, 0)),
            pl.BlockSpec((bm, 1), lambda i: (i, 0)),
            pl.BlockSpec((1, h_dim), lambda i: (0, 0)),
            pl.BlockSpec((h_dim, h_dim), lambda i: (0, 0)),
        ],
        out_specs=pl.BlockSpec((bm, h_dim), lambda i: (i, 0)),
        out_shape=jax.ShapeDtypeStruct((n_pad, h_dim), f32),
    )(s1p, y1, dinv, b1.reshape(1, h_dim), W2)

    s2p = seg_call(y2.reshape(-1), gidx, sidx, zeros4).reshape(NC, n_pad, 4)

    out, h2 = pl.pallas_call(
        _tc3_body,
        grid=(grid,),
        in_specs=[
            pl.BlockSpec((NC, bm, 4), lambda i: (0, i, 0)),
            pl.BlockSpec((bm, h_dim), lambda i: (i, 0)),
            pl.BlockSpec((bm, 1), lambda i: (i, 0)),
            pl.BlockSpec((1, h_dim), lambda i: (0, 0)),
            pl.BlockSpec((h_dim, c_dim), lambda i: (0, 0)),
            pl.BlockSpec((1, c_dim), lambda i: (0, 0)),
        ],
        out_specs=[
            pl.BlockSpec((bm, c_dim), lambda i: (i, 0)),
            pl.BlockSpec((bm, h_dim), lambda i: (i, 0)),
        ],
        out_shape=[
            jax.ShapeDtypeStruct((n, c_dim), f32),
            jax.ShapeDtypeStruct((n, h_dim), f32),
        ],
    )(s2p, y2, dinv, b2.reshape(1, h_dim), Wc, bc.reshape(1, c_dim))

    return (out, h2)


# speedup vs baseline: 41.8617x; 1.1015x over previous
import functools

import jax
import jax.numpy as jnp
from jax import lax
from jax.experimental import pallas as pl
from jax.experimental.pallas import tpu as pltpu
from jax.experimental.pallas import tpu_sc as plsc

NC = 2
NS = 16
CH = 128

f32 = jnp.float32
i32 = jnp.int32


def _mesh():
    return plsc.VectorSubcoreMesh(core_axis_name="c", subcore_axis_name="s",
                                  num_cores=NC, num_subcores=NS)


def _deg_body(colp, zeros, out, idx_v, ones_v, deg_sh, *, n_pad, k):
    c = lax.axis_index("c")
    s = lax.axis_index("s")
    sl = n_pad // NS
    pltpu.sync_copy(zeros.at[pl.ds(s * sl, sl)], deg_sh.at[pl.ds(s * sl, sl)])
    pltpu.sync_copy(colp.at[c].at[s], idx_v)
    for i in range(CH // 16):
        ones_v[pl.ds(i * 16, 16)] = jnp.full((16,), 1.0, f32)
    plsc.subcore_barrier()

    def body(j, _):
        pltpu.sync_copy(ones_v, deg_sh.at[idx_v.at[j]], add=True)
        return ()

    lax.fori_loop(0, k, body, ())
    plsc.subcore_barrier()
    pltpu.sync_copy(deg_sh.at[pl.ds(s * sl, sl)],
                    out.at[c].at[pl.ds(s * sl, sl)])


def _seg_body(yp, gidx, sidx, zeros, out, gidx_v, sidx_v, msg_v, y_sh, s_sh,
              gsem, ssem, *, n_pad, k4):
    c = lax.axis_index("c")
    s = lax.axis_index("s")
    k = k4 // 4
    sl = (n_pad * 4) // NS
    pltpu.sync_copy(zeros.at[pl.ds(s * sl, sl)], s_sh.at[pl.ds(s * sl, sl)])
    pltpu.sync_copy(yp.at[pl.ds(s * sl, sl)], y_sh.at[pl.ds(s * sl, sl)])
    for f in range(4):
        pltpu.sync_copy(gidx.at[f].at[c].at[s], gidx_v.at[f])
        pltpu.sync_copy(sidx.at[f].at[c].at[s], sidx_v.at[f])
    plsc.subcore_barrier()

    def gidx_at(j):
        return gidx_v.at[j // k].at[lax.rem(j, k)]

    def sidx_at(j):
        return sidx_v.at[j // k].at[lax.rem(j, k)]

    pltpu.async_copy(y_sh.at[gidx_at(0)], msg_v.at[0], gsem)
    pltpu.async_copy(y_sh.at[gidx_at(1)], msg_v.at[1], gsem)

    def body(j, _):
        b = lax.rem(j, 4)
        bn = lax.rem(j + 2, 4)

        @pl.when(j >= 2)
        def _():
            pltpu.make_async_copy(
                msg_v.at[bn], s_sh.at[sidx_at(j - 2)], ssem).wait()

        @pl.when(j + 2 < k4)
        def _():
            pltpu.async_copy(y_sh.at[gidx_at(j + 2)], msg_v.at[bn], gsem)

        pltpu.make_async_copy(y_sh.at[gidx_at(j)], msg_v.at[b], gsem).wait()
        pltpu.async_copy(msg_v.at[b], s_sh.at[sidx_at(j)], ssem, add=True)
        return ()

    lax.fori_loop(0, k4, body, ())
    pltpu.make_async_copy(
        msg_v.at[lax.rem(k4 - 2, 4)], s_sh.at[sidx_at(k4 - 2)], ssem).wait()
    pltpu.make_async_copy(
        msg_v.at[lax.rem(k4 - 1, 4)], s_sh.at[sidx_at(k4 - 1)], ssem).wait()
    plsc.subcore_barrier()
    pltpu.sync_copy(s_sh.at[pl.ds(s * sl, sl)],
                    out.at[c].at[pl.ds(s * sl, sl)])


def _tc1_body(x_ref, w1_ref, degp_ref, y_ref, dinv_ref):
    deg = degp_ref[0] + degp_ref[1] + 1.0
    dv = lax.rsqrt(deg)
    xw = jnp.dot(x_ref[...], w1_ref[...], preferred_element_type=f32)
    y_ref[...] = xw * dv
    dinv_ref[...] = dv


def _tc2_body(sp_ref, y_ref, dinv_ref, b_ref, w_ref, y2_ref):
    s = sp_ref[0] + sp_ref[1]
    dv = dinv_ref[...]
    h = jnp.tanh(dv * (s + y_ref[...]) + b_ref[...])
    y2_ref[...] = jnp.dot(h, w_ref[...], preferred_element_type=f32) * dv


def _tc3_body(sp_ref, y_ref, dinv_ref, b_ref, wc_ref, bc_ref,
              out_ref, h_ref):
    s = sp_ref[0] + sp_ref[1]
    dv = dinv_ref[...]
    h = jnp.tanh(dv * (s + y_ref[...]) + b_ref[...])
    h_ref[...] = h
    out_ref[...] = jnp.dot(h, wc_ref[...], preferred_element_type=f32) + bc_ref[...]


def kernel(x, edge_index, W1, b1, W2, b2, Wc, bc):
    n, d = x.shape
    h_dim = W1.shape[1]
    c_dim = Wc.shape[1]
    e = edge_index.shape[1]

    n_pad = ((n // (NS * 128)) + 1) * (NS * 128)
    k = -(-e // (NC * NS * CH))
    e_pad = NC * NS * k * CH

    row = edge_index[0]
    col = edge_index[1]
    pad = e_pad - e
    colp = jnp.concatenate(
        [col, n + (jnp.arange(pad, dtype=i32) % (n_pad - n))])
    colp = colp.reshape(NC, NS, k, CH)

    rowp = jnp.concatenate(
        [row, (jnp.arange(pad, dtype=i32) % n)]).reshape(NC, NS, k, CH)
    four = jnp.arange(4, dtype=i32).reshape(4, 1, 1, 1, 1)
    gidx = rowp[None] * 4 + four
    sidx = colp[None] * 4 + four
    k4 = 4 * k

    zeros1 = jnp.zeros((n_pad,), f32)
    zeros4 = jnp.zeros((n_pad * 4,), f32)

    mesh = _mesh()

    deg_call = pl.kernel(
        functools.partial(_deg_body, n_pad=n_pad, k=k),
        out_type=jax.ShapeDtypeStruct((NC, n_pad), f32),
        mesh=mesh,
        scratch_types=[
            pltpu.VMEM((k, CH), i32),
            pltpu.VMEM((CH,), f32),
            pltpu.VMEM_SHARED((n_pad,), f32),
        ],
    )
    degp = deg_call(colp, zeros1)

    seg_call = pl.kernel(
        functools.partial(_seg_body, n_pad=n_pad, k4=k4),
        out_type=jax.ShapeDtypeStruct((NC, n_pad * 4), f32),
        mesh=mesh,
        scratch_types=[
            pltpu.VMEM((4, k, CH), i32),
            pltpu.VMEM((4, k, CH), i32),
            pltpu.VMEM((4, CH), f32),
            pltpu.VMEM_SHARED((n_pad * 4,), f32),
            pltpu.VMEM_SHARED((n_pad * 4,), f32),
            pltpu.SemaphoreType.DMA,
            pltpu.SemaphoreType.DMA,
        ],
    )

    bm = 2000
    grid = n // bm
    degp3 = degp.reshape(NC, n_pad, 1)

    y1, dinv = pl.pallas_call(
        _tc1_body,
        grid=(grid,),
        in_specs=[
            pl.BlockSpec((bm, d), lambda i: (i, 0)),
            pl.BlockSpec((d, h_dim), lambda i: (0, 0)),
            pl.BlockSpec((NC, bm, 1), lambda i: (0, i, 0)),
        ],
        out_specs=[
            pl.BlockSpec((bm, h_dim), lambda i: (i, 0)),
            pl.BlockSpec((bm, 1), lambda i: (i, 0)),
        ],
        out_shape=[
            jax.ShapeDtypeStruct((n_pad, h_dim), f32),
            jax.ShapeDtypeStruct((n, 1), f32),
        ],
    )(x, W1, degp3)

    s1p = seg_call(y1.reshape(-1), gidx, sidx, zeros4).reshape(NC, n_pad, 4)

    y2 = pl.pallas_call(
        _tc2_body,
        grid=(grid,),
        in_specs=[
            pl.BlockSpec((NC, bm, 4), lambda i: (0, i, 0)),
            pl.BlockSpec((bm, h_dim), lambda i: (i, 0)),
            pl.BlockSpec((bm, 1), lambda i: (i, 0)),
            pl.BlockSpec((1, h_dim), lambda i: (0, 0)),
            pl.BlockSpec((h_dim, h_dim), lambda i: (0, 0)),
        ],
        out_specs=pl.BlockSpec((bm, h_dim), lambda i: (i, 0)),
        out_shape=jax.ShapeDtypeStruct((n_pad, h_dim), f32),
    )(s1p, y1, dinv, b1.reshape(1, h_dim), W2)

    s2p = seg_call(y2.reshape(-1), gidx, sidx, zeros4).reshape(NC, n_pad, 4)

    out, h2 = pl.pallas_call(
        _tc3_body,
        grid=(grid,),
        in_specs=[
            pl.BlockSpec((NC, bm, 4), lambda i: (0, i, 0)),
            pl.BlockSpec((bm, h_dim), lambda i: (i, 0)),
            pl.BlockSpec((bm, 1), lambda i: (i, 0)),
            pl.BlockSpec((1, h_dim), lambda i: (0, 0)),
            pl.BlockSpec((h_dim, c_dim), lambda i: (0, 0)),
            pl.BlockSpec((1, c_dim), lambda i: (0, 0)),
        ],
        out_specs=[
            pl.BlockSpec((bm, c_dim), lambda i: (i, 0)),
            pl.BlockSpec((bm, h_dim), lambda i: (i, 0)),
        ],
        out_shape=[
            jax.ShapeDtypeStruct((n, c_dim), f32),
            jax.ShapeDtypeStruct((n, h_dim), f32),
        ],
    )(s2p, y2, dinv, b2.reshape(1, h_dim), Wc, bc.reshape(1, c_dim))

    return (out, h2)


# speedup vs baseline: 59.7037x; 1.4262x over previous
import functools

import jax
import jax.numpy as jnp
from jax import lax
from jax.experimental import pallas as pl
from jax.experimental.pallas import tpu as pltpu
from jax.experimental.pallas import tpu_sc as plsc

NC = 2
NS = 16
CH = 128

f32 = jnp.float32
i32 = jnp.int32


def _mesh():
    return plsc.VectorSubcoreMesh(core_axis_name="c", subcore_axis_name="s",
                                  num_cores=NC, num_subcores=NS)


def _deg_body(colp, zeros, out, idx_v, ones_v, deg_sh, *, n_pad, k):
    c = lax.axis_index("c")
    s = lax.axis_index("s")
    sl = n_pad // NS
    pltpu.sync_copy(zeros.at[pl.ds(s * sl, sl)], deg_sh.at[pl.ds(s * sl, sl)])
    pltpu.sync_copy(colp.at[c].at[s], idx_v)
    for i in range(CH // 16):
        ones_v[pl.ds(i * 16, 16)] = jnp.full((16,), 1.0, f32)
    plsc.subcore_barrier()

    def body(j, _):
        pltpu.sync_copy(ones_v, deg_sh.at[idx_v.at[j]], add=True)
        return ()

    lax.fori_loop(0, k, body, ())
    plsc.subcore_barrier()
    pltpu.sync_copy(deg_sh.at[pl.ds(s * sl, sl)],
                    out.at[c].at[pl.ds(s * sl, sl)])


def _seg_body(yp, gidx, sidx, zeros, out, gidx_v, sidx_v, msg_v, y_sh, s_sh,
              gsem, ssem, *, n_pad, k4):
    c = lax.axis_index("c")
    s = lax.axis_index("s")
    k = k4 // 4
    sl = (n_pad * 4) // NS
    sl1 = n_pad // NS
    pltpu.sync_copy(zeros.at[pl.ds(s * sl, sl)], s_sh.at[pl.ds(s * sl, sl)])
    for f in range(4):
        pltpu.sync_copy(yp.at[f].at[pl.ds(s * sl1, sl1)],
                        y_sh.at[pl.ds(f * n_pad + s * sl1, sl1)])
        pltpu.sync_copy(gidx.at[f].at[c].at[s], gidx_v.at[f])
        pltpu.sync_copy(sidx.at[f].at[c].at[s], sidx_v.at[f])
    plsc.subcore_barrier()

    def gidx_at(j):
        return gidx_v.at[j // k].at[lax.rem(j, k)]

    def sidx_at(j):
        return sidx_v.at[j // k].at[lax.rem(j, k)]

    for g in range(4):
        pltpu.async_copy(y_sh.at[gidx_at(g)], msg_v.at[g], gsem)

    def body(j, _):
        b = lax.rem(j, 8)
        bn = lax.rem(j + 4, 8)

        @pl.when(j >= 4)
        def _():
            pltpu.make_async_copy(
                msg_v.at[bn], s_sh.at[sidx_at(j - 4)], ssem).wait()

        @pl.when(j + 4 < k4)
        def _():
            pltpu.async_copy(y_sh.at[gidx_at(j + 4)], msg_v.at[bn], gsem)

        pltpu.make_async_copy(y_sh.at[gidx_at(j)], msg_v.at[b], gsem).wait()
        pltpu.async_copy(msg_v.at[b], s_sh.at[sidx_at(j)], ssem, add=True)
        return ()

    lax.fori_loop(0, k4, body, ())
    for g in range(4):
        pltpu.make_async_copy(
            msg_v.at[lax.rem(k4 - 4 + g, 8)],
            s_sh.at[sidx_at(k4 - 4 + g)], ssem).wait()
    plsc.subcore_barrier()
    for f in range(4):
        pltpu.sync_copy(s_sh.at[pl.ds(f * n_pad + s * sl1, sl1)],
                        out.at[c].at[f].at[pl.ds(s * sl1, sl1)])


_TDIMS = (((0,), (0,)), ((), ()))


def _tc1_body(x_ref, w1_ref, degp_ref, y_ref, dinv_ref):
    deg = degp_ref[0:1, :] + degp_ref[1:2, :] + 1.0
    dv = lax.rsqrt(deg)
    xwt = lax.dot_general(w1_ref[...], x_ref[...], (((0,), (1,)), ((), ())),
                          preferred_element_type=f32)
    y_ref[...] = xwt * dv
    dinv_ref[...] = dv


def _tc2_body(sp_ref, y_ref, dinv_ref, b_ref, w_ref, y2_ref):
    s = sp_ref[0] + sp_ref[1]
    dv = dinv_ref[...]
    h = jnp.tanh(dv * (s + y_ref[...]) + b_ref[...])
    y2_ref[...] = lax.dot_general(w_ref[...], h, _TDIMS,
                                  preferred_element_type=f32) * dv


def _tc3_body(sp_ref, y_ref, dinv_ref, b_ref, wc_ref, bc_ref,
              out_ref, h_ref):
    s = sp_ref[0] + sp_ref[1]
    dv = dinv_ref[...]
    ht = jnp.tanh(dv * (s + y_ref[...]) + b_ref[...])
    h = ht.T
    h_ref[...] = h
    out_ref[...] = jnp.dot(h, wc_ref[...], preferred_element_type=f32) + bc_ref[...]


def kernel(x, edge_index, W1, b1, W2, b2, Wc, bc):
    n, d = x.shape
    h_dim = W1.shape[1]
    c_dim = Wc.shape[1]
    e = edge_index.shape[1]

    n_pad = ((n // (NS * 128)) + 1) * (NS * 128)
    k = -(-e // (NC * NS * CH))
    e_pad = NC * NS * k * CH

    row = edge_index[0]
    col = edge_index[1]
    pad = e_pad - e
    colp = jnp.concatenate(
        [col, n + (jnp.arange(pad, dtype=i32) % (n_pad - n))])
    colp = colp.reshape(NC, NS, k, CH)

    rowp = jnp.concatenate(
        [row, (jnp.arange(pad, dtype=i32) % n)]).reshape(NC, NS, k, CH)
    four = jnp.arange(4, dtype=i32).reshape(4, 1, 1, 1, 1)
    gidx = rowp[None] + four * n_pad
    sidx = colp[None] + four * n_pad
    k4 = 4 * k

    zeros1 = jnp.zeros((n_pad,), f32)
    zeros4 = jnp.zeros((n_pad * 4,), f32)

    mesh = _mesh()

    deg_call = pl.kernel(
        functools.partial(_deg_body, n_pad=n_pad, k=k),
        out_type=jax.ShapeDtypeStruct((NC, n_pad), f32),
        mesh=mesh,
        scratch_types=[
            pltpu.VMEM((k, CH), i32),
            pltpu.VMEM((CH,), f32),
            pltpu.VMEM_SHARED((n_pad,), f32),
        ],
    )
    degp = deg_call(colp, zeros1)

    seg_call = pl.kernel(
        functools.partial(_seg_body, n_pad=n_pad, k4=k4),
        out_type=jax.ShapeDtypeStruct((NC, 4, n_pad), f32),
        mesh=mesh,
        scratch_types=[
            pltpu.VMEM((4, k, CH), i32),
            pltpu.VMEM((4, k, CH), i32),
            pltpu.VMEM((8, CH), f32),
            pltpu.VMEM_SHARED((n_pad * 4,), f32),
            pltpu.VMEM_SHARED((n_pad * 4,), f32),
            pltpu.SemaphoreType.DMA,
            pltpu.SemaphoreType.DMA,
        ],
    )

    bm = 2048
    grid = n_pad // bm

    y1, dinv = pl.pallas_call(
        _tc1_body,
        grid=(grid,),
        in_specs=[
            pl.BlockSpec((bm, d), lambda i: (i, 0)),
            pl.BlockSpec((d, h_dim), lambda i: (0, 0)),
            pl.BlockSpec((NC, bm), lambda i: (0, i)),
        ],
        out_specs=[
            pl.BlockSpec((h_dim, bm), lambda i: (0, i)),
            pl.BlockSpec((1, bm), lambda i: (0, i)),
        ],
        out_shape=[
            jax.ShapeDtypeStruct((h_dim, n_pad), f32),
            jax.ShapeDtypeStruct((1, n_pad), f32),
        ],
    )(x, W1, degp)

    s1p = seg_call(y1, gidx, sidx, zeros4)

    y2 = pl.pallas_call(
        _tc2_body,
        grid=(grid,),
        in_specs=[
            pl.BlockSpec((NC, h_dim, bm), lambda i: (0, 0, i)),
            pl.BlockSpec((h_dim, bm), lambda i: (0, i)),
            pl.BlockSpec((1, bm), lambda i: (0, i)),
            pl.BlockSpec((h_dim, 1), lambda i: (0, 0)),
            pl.BlockSpec((h_dim, h_dim), lambda i: (0, 0)),
        ],
        out_specs=pl.BlockSpec((h_dim, bm), lambda i: (0, i)),
        out_shape=jax.ShapeDtypeStruct((h_dim, n_pad), f32),
    )(s1p, y1, dinv, b1.reshape(h_dim, 1), W2)

    s2p = seg_call(y2, gidx, sidx, zeros4)

    out, h2 = pl.pallas_call(
        _tc3_body,
        grid=(grid,),
        in_specs=[
            pl.BlockSpec((NC, h_dim, bm), lambda i: (0, 0, i)),
            pl.BlockSpec((h_dim, bm), lambda i: (0, i)),
            pl.BlockSpec((1, bm), lambda i: (0, i)),
            pl.BlockSpec((h_dim, 1), lambda i: (0, 0)),
            pl.BlockSpec((h_dim, c_dim), lambda i: (0, 0)),
            pl.BlockSpec((1, c_dim), lambda i: (0, 0)),
        ],
        out_specs=[
            pl.BlockSpec((bm, c_dim), lambda i: (i, 0)),
            pl.BlockSpec((bm, h_dim), lambda i: (i, 0)),
        ],
        out_shape=[
            jax.ShapeDtypeStruct((n, c_dim), f32),
            jax.ShapeDtypeStruct((n, h_dim), f32),
        ],
    )(s2p, y2, dinv, b2.reshape(h_dim, 1), Wc, bc.reshape(1, c_dim))

    return (out, h2)


# speedup vs baseline: 62.2591x; 1.0428x over previous
import functools

import jax
import jax.numpy as jnp
from jax import lax
from jax.experimental import pallas as pl
from jax.experimental.pallas import tpu as pltpu
from jax.experimental.pallas import tpu_sc as plsc

NC = 2
NS = 16
CH = 128

f32 = jnp.float32
i32 = jnp.int32


def _mesh():
    return plsc.VectorSubcoreMesh(core_axis_name="c", subcore_axis_name="s",
                                  num_cores=NC, num_subcores=NS)


def _deg_body(colp, zeros, out, idx_v, ones_v, deg_sh, *, n_pad, k):
    c = lax.axis_index("c")
    s = lax.axis_index("s")
    sl = n_pad // NS
    pltpu.sync_copy(zeros.at[pl.ds(s * sl, sl)], deg_sh.at[pl.ds(s * sl, sl)])
    pltpu.sync_copy(colp.at[c].at[s], idx_v)
    for i in range(CH // 16):
        ones_v[pl.ds(i * 16, 16)] = jnp.full((16,), 1.0, f32)
    plsc.subcore_barrier()

    def body(j, _):
        pltpu.sync_copy(ones_v, deg_sh.at[idx_v.at[j]], add=True)
        return ()

    lax.fori_loop(0, k, body, ())
    plsc.subcore_barrier()
    pltpu.sync_copy(deg_sh.at[pl.ds(s * sl, sl)],
                    out.at[c].at[pl.ds(s * sl, sl)])


def _seg_body(yp, gidx, sidx, zeros, out, gidx_v, sidx_v, msg_v, y_sh, s_sh,
              gsem, ssem, *, n_pad, k4):
    c = lax.axis_index("c")
    s = lax.axis_index("s")
    k = k4 // 4
    sl = (n_pad * 4) // NS
    sl1 = n_pad // NS
    pltpu.sync_copy(zeros.at[pl.ds(s * sl, sl)], s_sh.at[pl.ds(s * sl, sl)])
    for f in range(4):
        pltpu.sync_copy(yp.at[f].at[pl.ds(s * sl1, sl1)],
                        y_sh.at[pl.ds(f * n_pad + s * sl1, sl1)])
    pltpu.sync_copy(gidx.at[c].at[s], gidx_v)
    pltpu.sync_copy(sidx.at[c].at[s], sidx_v)
    plsc.subcore_barrier()

    def gidx_at(j):
        return gidx_v.at[j]

    def sidx_at(j):
        return sidx_v.at[j]

    for g in range(4):
        pltpu.async_copy(y_sh.at[gidx_at(g)], msg_v.at[g], gsem)

    def body(j, _):
        b = lax.rem(j, 8)
        bn = lax.rem(j + 4, 8)

        @pl.when(j >= 4)
        def _():
            pltpu.make_async_copy(
                msg_v.at[bn], s_sh.at[sidx_at(j - 4)], ssem).wait()

        @pl.when(j + 4 < k4)
        def _():
            pltpu.async_copy(y_sh.at[gidx_at(j + 4)], msg_v.at[bn], gsem)

        pltpu.make_async_copy(y_sh.at[gidx_at(j)], msg_v.at[b], gsem).wait()
        pltpu.async_copy(msg_v.at[b], s_sh.at[sidx_at(j)], ssem, add=True)
        return ()

    lax.fori_loop(0, k4, body, ())
    for g in range(4):
        pltpu.make_async_copy(
            msg_v.at[lax.rem(k4 - 4 + g, 8)],
            s_sh.at[sidx_at(k4 - 4 + g)], ssem).wait()
    plsc.subcore_barrier()
    for f in range(4):
        pltpu.sync_copy(s_sh.at[pl.ds(f * n_pad + s * sl1, sl1)],
                        out.at[c].at[f].at[pl.ds(s * sl1, sl1)])


_TDIMS = (((0,), (0,)), ((), ()))


def _tc1_body(x_ref, w1_ref, degp_ref, y_ref, dinv_ref):
    deg = degp_ref[0:1, :] + degp_ref[1:2, :] + 1.0
    dv = lax.rsqrt(deg)
    xwt = lax.dot_general(w1_ref[...], x_ref[...], (((0,), (1,)), ((), ())),
                          preferred_element_type=f32)
    y_ref[...] = xwt * dv
    dinv_ref[...] = dv


def _tc2_body(sp_ref, y_ref, dinv_ref, b_ref, w_ref, y2_ref):
    s = sp_ref[0] + sp_ref[1]
    dv = dinv_ref[...]
    h = jnp.tanh(dv * (s + y_ref[...]) + b_ref[...])
    y2_ref[...] = lax.dot_general(w_ref[...], h, _TDIMS,
                                  preferred_element_type=f32) * dv


def _tc3_body(sp_ref, y_ref, dinv_ref, b_ref, wc_ref, bc_ref,
              out_ref, h_ref):
    s = sp_ref[0] + sp_ref[1]
    dv = dinv_ref[...]
    ht = jnp.tanh(dv * (s + y_ref[...]) + b_ref[...])
    h = ht.T
    h_ref[...] = h
    out_ref[...] = jnp.dot(h, wc_ref[...], preferred_element_type=f32) + bc_ref[...]


def kernel(x, edge_index, W1, b1, W2, b2, Wc, bc):
    n, d = x.shape
    h_dim = W1.shape[1]
    c_dim = Wc.shape[1]
    e = edge_index.shape[1]

    n_pad = ((n // (NS * 128)) + 1) * (NS * 128)
    k = -(-e // (NC * NS * CH))
    k = ((k + 7) // 8) * 8
    e_pad = NC * NS * k * CH

    row = edge_index[0]
    col = edge_index[1]
    pad = e_pad - e
    colp = jnp.concatenate(
        [col, n + (jnp.arange(pad, dtype=i32) % (n_pad - n))])
    colp = colp.reshape(NC, NS, k, CH)

    rowp = jnp.concatenate(
        [row, (jnp.arange(pad, dtype=i32) % n)]).reshape(NC, NS, k, CH)
    four = jnp.arange(4, dtype=i32).reshape(1, 1, 4, 1, 1)
    k4 = 4 * k
    gidx = (rowp[:, :, None] + four * n_pad).reshape(NC, NS, k4, CH)
    sidx = (colp[:, :, None] + four * n_pad).reshape(NC, NS, k4, CH)

    zeros1 = jnp.zeros((n_pad,), f32)
    zeros4 = jnp.zeros((n_pad * 4,), f32)

    mesh = _mesh()

    deg_call = pl.kernel(
        functools.partial(_deg_body, n_pad=n_pad, k=k),
        out_type=jax.ShapeDtypeStruct((NC, n_pad), f32),
        mesh=mesh,
        scratch_types=[
            pltpu.VMEM((k, CH), i32),
            pltpu.VMEM((CH,), f32),
            pltpu.VMEM_SHARED((n_pad,), f32),
        ],
    )
    degp = deg_call(colp, zeros1)

    seg_call = pl.kernel(
        functools.partial(_seg_body, n_pad=n_pad, k4=k4),
        out_type=jax.ShapeDtypeStruct((NC, 4, n_pad), f32),
        mesh=mesh,
        scratch_types=[
            pltpu.VMEM((k4, CH), i32),
            pltpu.VMEM((k4, CH), i32),
            pltpu.VMEM((8, CH), f32),
            pltpu.VMEM_SHARED((n_pad * 4,), f32),
            pltpu.VMEM_SHARED((n_pad * 4,), f32),
            pltpu.SemaphoreType.DMA,
            pltpu.SemaphoreType.DMA,
        ],
    )

    bm = 2048
    grid = n_pad // bm

    y1, dinv = pl.pallas_call(
        _tc1_body,
        grid=(grid,),
        in_specs=[
            pl.BlockSpec((bm, d), lambda i: (i, 0)),
            pl.BlockSpec((d, h_dim), lambda i: (0, 0)),
            pl.BlockSpec((NC, bm), lambda i: (0, i)),
        ],
        out_specs=[
            pl.BlockSpec((h_dim, bm), lambda i: (0, i)),
            pl.BlockSpec((1, bm), lambda i: (0, i)),
        ],
        out_shape=[
            jax.ShapeDtypeStruct((h_dim, n_pad), f32),
            jax.ShapeDtypeStruct((1, n_pad), f32),
        ],
    )(x, W1, degp)

    s1p = seg_call(y1, gidx, sidx, zeros4)

    y2 = pl.pallas_call(
        _tc2_body,
        grid=(grid,),
        in_specs=[
            pl.BlockSpec((NC, h_dim, bm), lambda i: (0, 0, i)),
            pl.BlockSpec((h_dim, bm), lambda i: (0, i)),
            pl.BlockSpec((1, bm), lambda i: (0, i)),
            pl.BlockSpec((h_dim, 1), lambda i: (0, 0)),
            pl.BlockSpec((h_dim, h_dim), lambda i: (0, 0)),
        ],
        out_specs=pl.BlockSpec((h_dim, bm), lambda i: (0, i)),
        out_shape=jax.ShapeDtypeStruct((h_dim, n_pad), f32),
    )(s1p, y1, dinv, b1.reshape(h_dim, 1), W2)

    s2p = seg_call(y2, gidx, sidx, zeros4)

    out, h2 = pl.pallas_call(
        _tc3_body,
        grid=(grid,),
        in_specs=[
            pl.BlockSpec((NC, h_dim, bm), lambda i: (0, 0, i)),
            pl.BlockSpec((h_dim, bm), lambda i: (0, i)),
            pl.BlockSpec((1, bm), lambda i: (0, i)),
            pl.BlockSpec((h_dim, 1), lambda i: (0, 0)),
            pl.BlockSpec((h_dim, c_dim), lambda i: (0, 0)),
            pl.BlockSpec((1, c_dim), lambda i: (0, 0)),
        ],
        out_specs=[
            pl.BlockSpec((bm, c_dim), lambda i: (i, 0)),
            pl.BlockSpec((bm, h_dim), lambda i: (i, 0)),
        ],
        out_shape=[
            jax.ShapeDtypeStruct((n, c_dim), f32),
            jax.ShapeDtypeStruct((n, h_dim), f32),
        ],
    )(s2p, y2, dinv, b2.reshape(h_dim, 1), Wc, bc.reshape(1, c_dim))

    return (out, h2)
